# baseline (speedup 1.0000x reference)
import jax
import jax.numpy as jnp
from jax import lax
from jax.experimental import pallas as pl
from jax.experimental.pallas import tpu as pltpu
from jax.experimental.pallas import tpu_sc as plsc

N = 50000
D = 256
S = 512
NLANE = 16
NT = D // NLANE
NW = 32
SEG_PER_W = S // NW
CHUNK = 128


def _body(x_hbm, starts_hbm, wb_hbm, out_hbm, starts_v, wb_v, xbuf, gbuf,
          acc_v):
    c = lax.axis_index("c")
    s = lax.axis_index("s")
    w = s * 2 + c
    base = SEG_PER_W * w

    pltpu.sync_copy(starts_hbm, starts_v)
    pltpu.sync_copy(wb_hbm, wb_v)

    lanes = lax.iota(jnp.int32, NLANE)
    va = starts_v[pl.ds(base, NLANE)]
    vb = starts_v[pl.ds(base + 1, NLANE)]
    r0_all = va[0]
    r1_all = starts_v[pl.ds(base + SEG_PER_W, NLANE)][0]
    bvec = wb_v[pl.ds(D, NLANE)]
    wv = [wb_v[pl.ds(NLANE * t, NLANE)] for t in range(NT)]

    zero16 = jnp.zeros((NLANE,), jnp.float32)
    ninf16 = jnp.full((NLANE,), -jnp.inf, jnp.float32)
    for k in range(SEG_PER_W):
        for t in range(NT):
            acc_v[k, pl.ds(NLANE * t, NLANE)] = zero16
            acc_v[k, pl.ds(D + NLANE * t, NLANE)] = ninf16

    c0 = (r0_all // 8) * 8
    nchunks = (r1_all - c0 + CHUNK - 1) // CHUNK

    @pl.loop(0, nchunks)
    def _chunk(i):
        s_i = jnp.minimum(c0 + i * CHUNK, N - CHUNK)
        s_i = pl.multiple_of(s_i, 8)
        pltpu.sync_copy(x_hbm.at[pl.ds(s_i, CHUNK)],
                        xbuf.at[pl.ds(0, CHUNK)])
        lo = jnp.maximum(r0_all, c0 + i * CHUNK)
        hi = jnp.minimum(r1_all, c0 + (i + 1) * CHUNK)
        hi = jnp.maximum(hi, lo)

        nga = (hi - lo + 3) // 4

        @pl.loop(0, nga)
        def _row_a(i4):
            j0 = lo + 4 * i4 - s_i
            for u in range(4):
                jj = j0 + u
                xs = [xbuf[jj, pl.ds(NLANE * t, NLANE)] for t in range(NT)]
                p = [xs[t] * wv[t] for t in range(NT)]
                while len(p) > 1:
                    p = [p[i2] + p[i2 + 1] for i2 in range(0, len(p), 2)]
                zv = p[0]
                for m in (8, 4, 2, 1):
                    zv = zv + zv.at[lanes ^ m].get(
                        mode="promise_in_bounds", unique_indices=True)
                gbuf[jj] = 1.0 / (1.0 + jnp.exp(-(zv + bvec)))

        def _bsum_i32(v):
            for m in (8, 4, 2, 1):
                v = v + v.at[lanes ^ m].get(
                    mode="promise_in_bounds", unique_indices=True)
            return v

        ks = _bsum_i32(jnp.where(vb <= lo, 1, 0))[0]
        ke = _bsum_i32(jnp.where(va < hi, 1, 0))[0]

        @pl.loop(ks, ke)
        def _seg(k):
            b0 = starts_v[pl.ds(base + k, NLANE)][0]
            b1 = starts_v[pl.ds(base + k + 1, NLANE)][0]
            a = jnp.maximum(b0, lo)
            e = jnp.minimum(b1, hi)
            e = jnp.maximum(e, a)
            sacc = tuple(acc_v[k, pl.ds(NLANE * t, NLANE)]
                         for t in range(NT))
            macc = tuple(acc_v[k, pl.ds(D + NLANE * t, NLANE)]
                         for t in range(NT))

            ngb = (e - a + 1) // 2

            @pl.loop(0, ngb, init_carry=(sacc, macc))
            def _row_b(i2, carry):
                sa, ma = carry
                jj = a + 2 * i2 - s_i
                g0 = gbuf[jj]
                x0 = [xbuf[jj, pl.ds(NLANE * t, NLANE)] for t in range(NT)]
                valid = (a + 2 * i2 + 1) < e
                g1 = jnp.where(valid, gbuf[jj + 1], zero16)
                x1 = [xbuf[jj + 1, pl.ds(NLANE * t, NLANE)]
                      for t in range(NT)]
                x1m = [jnp.where(valid, x1[t], ninf16) for t in range(NT)]
                x1z = [jnp.where(valid, x1[t], zero16) for t in range(NT)]
                sa = tuple(sa[t] + g0 * x0[t] + g1 * x1z[t]
                           for t in range(NT))
                ma = tuple(jnp.maximum(jnp.maximum(ma[t], x0[t]), x1m[t])
                           for t in range(NT))
                return (sa, ma)

            sacc, macc = _row_b
            for t in range(NT):
                acc_v[k, pl.ds(NLANE * t, NLANE)] = sacc[t]
                acc_v[k, pl.ds(D + NLANE * t, NLANE)] = macc[t]

    pltpu.sync_copy(acc_v, out_hbm.at[pl.ds(SEG_PER_W * w, SEG_PER_W)])


_mesh = plsc.VectorSubcoreMesh(core_axis_name="c", subcore_axis_name="s")

_sc_call = pl.kernel(
    _body,
    out_type=jax.ShapeDtypeStruct((S, 2 * D), jnp.float32),
    mesh=_mesh,
    scratch_types=[
        pltpu.VMEM((544,), jnp.int32),
        pltpu.VMEM((272,), jnp.float32),
        pltpu.VMEM((CHUNK + 4, D), jnp.float32),
        pltpu.VMEM((CHUNK + 4, NLANE), jnp.float32),
        pltpu.VMEM((SEG_PER_W, 2 * D), jnp.float32),
    ],
)


def kernel(x, batch, W, b):
    batch32 = batch.astype(jnp.int32)
    ids = jnp.arange(S + 1, dtype=jnp.int32)
    starts = jnp.searchsorted(batch32, ids).astype(jnp.int32)
    starts = jnp.concatenate([starts, jnp.zeros((31,), jnp.int32)])
    wb = jnp.concatenate([
        W.reshape(-1).astype(jnp.float32),
        jnp.broadcast_to(b.astype(jnp.float32), (16,)),
    ])
    return _sc_call(x, starts, wb)


# speedup vs baseline: 1.1243x; 1.1243x over previous
import jax
import jax.numpy as jnp
from jax import lax
from jax.experimental import pallas as pl
from jax.experimental.pallas import tpu as pltpu
from jax.experimental.pallas import tpu_sc as plsc

N = 50000
D = 256
S = 512
NLANE = 16
NT = D // NLANE
NW = 32
SEG_PER_W = S // NW
CHUNK = 124
XLEN = (CHUNK + 4) * D


def _body(x_hbm, starts_hbm, wb_hbm, out_hbm, starts_v, wb_v, xbuf2, gbuf,
          acc_v, sems):
    c = lax.axis_index("c")
    s = lax.axis_index("s")
    w = s * 2 + c
    base = SEG_PER_W * w

    pltpu.sync_copy(starts_hbm, starts_v)
    pltpu.sync_copy(wb_hbm, wb_v)

    lanes = lax.iota(jnp.int32, NLANE)
    va = starts_v[pl.ds(base, NLANE)]
    vb = starts_v[pl.ds(base + 1, NLANE)]
    r0_all = va[0]
    r1_all = starts_v[pl.ds(base + SEG_PER_W, NLANE)][0]
    bvec = wb_v[pl.ds(D, NLANE)]
    wv = [wb_v[pl.ds(NLANE * t, NLANE)] for t in range(NT)]

    zero16 = jnp.zeros((NLANE,), jnp.float32)
    ninf16 = jnp.full((NLANE,), -jnp.inf, jnp.float32)
    for k in range(SEG_PER_W):
        for t in range(NT):
            acc_v[k, pl.ds(NLANE * t, NLANE)] = zero16
            acc_v[k, pl.ds(D + NLANE * t, NLANE)] = ninf16

    c0 = r0_all
    nchunks = (r1_all - c0 + CHUNK - 1) // CHUNK

    def _start(ci, parity):
        sj = jnp.minimum(c0 + ci * CHUNK, N - CHUNK)
        pltpu.async_copy(x_hbm.at[pl.ds(sj * D, CHUNK * D)],
                         xbuf2.at[parity, pl.ds(0, CHUNK * D)],
                         sems.at[parity])

    def _wait(ci, parity):
        sj = jnp.minimum(c0 + ci * CHUNK, N - CHUNK)
        pltpu.make_async_copy(x_hbm.at[pl.ds(sj * D, CHUNK * D)],
                              xbuf2.at[parity, pl.ds(0, CHUNK * D)],
                              sems.at[parity]).wait()

    _start(0, 0)

    @pl.loop(0, nchunks)
    def _chunk(i):
        p = jnp.bitwise_and(i, 1)
        s_i = jnp.minimum(c0 + i * CHUNK, N - CHUNK)
        _wait(i, p)
        _start(jnp.minimum(i + 1, nchunks - 1), 1 - p)
        xb = xbuf2.at[p]
        lo = jnp.maximum(r0_all, c0 + i * CHUNK)
        hi = jnp.minimum(r1_all, c0 + (i + 1) * CHUNK)
        hi = jnp.maximum(hi, lo)

        nga = (hi - lo + 3) // 4

        @pl.loop(0, nga)
        def _row_a(i4):
            j0 = lo + 4 * i4 - s_i
            for u in range(4):
                jj = j0 + u
                xs = [xb[pl.ds(jj * D + NLANE * t, NLANE)]
                      for t in range(NT)]
                pp = [xs[t] * wv[t] for t in range(NT)]
                while len(pp) > 1:
                    pp = [pp[i2] + pp[i2 + 1] for i2 in range(0, len(pp), 2)]
                zv = pp[0]
                for m in (8, 4, 2, 1):
                    zv = zv + zv.at[lanes ^ m].get(
                        mode="promise_in_bounds", unique_indices=True)
                gbuf[jj] = 1.0 / (1.0 + jnp.exp(-(zv + bvec)))

        def _bsum_i32(v):
            for m in (8, 4, 2, 1):
                v = v + v.at[lanes ^ m].get(
                    mode="promise_in_bounds", unique_indices=True)
            return v

        ks = _bsum_i32(jnp.where(vb <= lo, 1, 0))[0]
        ke = _bsum_i32(jnp.where(va < hi, 1, 0))[0]

        @pl.loop(ks, ke)
        def _seg(k):
            b0 = starts_v[pl.ds(base + k, NLANE)][0]
            b1 = starts_v[pl.ds(base + k + 1, NLANE)][0]
            a = jnp.maximum(b0, lo)
            e = jnp.minimum(b1, hi)
            e = jnp.maximum(e, a)
            sacc = tuple(acc_v[k, pl.ds(NLANE * t, NLANE)]
                         for t in range(NT))
            macc = tuple(acc_v[k, pl.ds(D + NLANE * t, NLANE)]
                         for t in range(NT))

            ngb = (e - a + 1) // 2

            @pl.loop(0, ngb, init_carry=(sacc, macc))
            def _row_b(i2, carry):
                sa, ma = carry
                jj = a + 2 * i2 - s_i
                g0 = gbuf[jj]
                x0 = [xb[pl.ds(jj * D + NLANE * t, NLANE)]
                      for t in range(NT)]
                valid = (a + 2 * i2 + 1) < e
                g1 = jnp.where(valid, gbuf[jj + 1], zero16)
                x1 = [xb[pl.ds((jj + 1) * D + NLANE * t, NLANE)]
                      for t in range(NT)]
                x1m = [jnp.where(valid, x1[t], ninf16) for t in range(NT)]
                x1z = [jnp.where(valid, x1[t], zero16) for t in range(NT)]
                sa = tuple(sa[t] + g0 * x0[t] + g1 * x1z[t]
                           for t in range(NT))
                ma = tuple(jnp.maximum(jnp.maximum(ma[t], x0[t]), x1m[t])
                           for t in range(NT))
                return (sa, ma)

            sacc, macc = _row_b
            for t in range(NT):
                acc_v[k, pl.ds(NLANE * t, NLANE)] = sacc[t]
                acc_v[k, pl.ds(D + NLANE * t, NLANE)] = macc[t]

    _wait(jnp.maximum(nchunks - 1, 0), jnp.bitwise_and(nchunks, 1))
    pltpu.sync_copy(acc_v, out_hbm.at[pl.ds(SEG_PER_W * w, SEG_PER_W)])


_mesh = plsc.VectorSubcoreMesh(core_axis_name="c", subcore_axis_name="s")

_sc_call = pl.kernel(
    _body,
    out_type=jax.ShapeDtypeStruct((S, 2 * D), jnp.float32),
    mesh=_mesh,
    scratch_types=[
        pltpu.VMEM((544,), jnp.int32),
        pltpu.VMEM((272,), jnp.float32),
        pltpu.VMEM((2, XLEN), jnp.float32),
        pltpu.VMEM((CHUNK + 4, NLANE), jnp.float32),
        pltpu.VMEM((SEG_PER_W, 2 * D), jnp.float32),
        pltpu.SemaphoreType.DMA((2,)),
    ],
)


def kernel(x, batch, W, b):
    batch32 = batch.astype(jnp.int32)
    ids = jnp.arange(S + 1, dtype=jnp.int32)
    starts = jnp.searchsorted(batch32, ids).astype(jnp.int32)
    starts = jnp.concatenate([starts, jnp.zeros((31,), jnp.int32)])
    wb = jnp.concatenate([
        W.reshape(-1).astype(jnp.float32),
        jnp.broadcast_to(b.astype(jnp.float32), (16,)),
    ])
    return _sc_call(x.reshape(-1), starts, wb)
